# Initial kernel scaffold; baseline (speedup 1.0000x reference)
import jax
import jax.numpy as jnp
from jax.experimental import pallas as pl


def kernel(in_flow, score):
    raise NotImplementedError("write your pallas kernel here")


# speedup vs baseline: 2.3182x; 2.3182x over previous
import functools

import jax
import jax.numpy as jnp
from jax import lax
from jax.experimental import pallas as pl
from jax.experimental.pallas import tpu as pltpu
from jax.experimental.pallas import tpu_sc as plsc

T = 32768
E = 64
D = 768

_info = plsc.get_sparse_core_info()
NC, NS, L = _info.num_cores, _info.num_subcores, _info.num_lanes
NW = NC * NS
TPW = T // NW
G = TPW // L
CH = 64
NCH = TPW // CH

_MESH = plsc.VectorSubcoreMesh(core_axis_name="c", subcore_axis_name="s")


def _wid():
    return lax.axis_index("s") * NC + lax.axis_index("c")


_CPARAMS = pltpu.CompilerParams(needs_layout_passes=False)


@functools.partial(
    pl.kernel,
    mesh=_MESH,
    compiler_params=_CPARAMS,
    out_type=(
        jax.ShapeDtypeStruct((T,), jnp.int32),
        jax.ShapeDtypeStruct((T,), jnp.int32),
        jax.ShapeDtypeStruct((NW, E), jnp.int32),
    ),
    scratch_types=[
        pltpu.VMEM((TPW * E,), jnp.float32),
        pltpu.VMEM((TPW,), jnp.int32),
        pltpu.VMEM((TPW,), jnp.int32),
        pltpu.VMEM((E,), jnp.int32),
        pltpu.VMEM((48,), jnp.int32),
    ],
)
def _routing(score_hbm, eidx_hbm, lrank_hbm, hist_hbm,
             score_v, eidx_v, lrank_v, run_v, tmp_v):
    wid = _wid()
    base = wid * TPW
    pltpu.sync_copy(score_hbm.at[pl.ds(base * E, TPW * E)], score_v)

    lane = lax.iota(jnp.int32, L)
    zeros = jnp.zeros((L,), jnp.int32)
    neg = jnp.full((L,), -1, jnp.int32)
    for q in range(E // L):
        run_v[pl.ds(q * L, L)] = zeros
    tmp_v[pl.ds(0, L)] = neg
    tmp_v[pl.ds(2 * L, L)] = neg

    def group_body(g, _):
        tbase = g * L
        gidx = (tbase + lane) * E
        bv0 = plsc.load_gather(score_v, [gidx])

        def am_body(e, carry):
            bv, bi = carry
            val = plsc.load_gather(score_v, [gidx + e])
            upd = val > bv
            return (jnp.where(upd, val, bv),
                    jnp.where(upd, jnp.full((L,), e, jnp.int32), bi))

        bv, bi = lax.fori_loop(1, E, am_body, (bv0, zeros))

        tmp_v[pl.ds(L, L)] = neg
        tmp_v[pl.ds(L - 1, L)] = bi
        prefix = zeros
        extra = zeros
        for s in range(1, L):
            sm = tmp_v[pl.ds(L - 1 - s, L)]
            sp = tmp_v[pl.ds(L - 1 + s, L)]
            prefix = prefix + (sm == bi).astype(jnp.int32)
            extra = extra + (sp == bi).astype(jnp.int32)

        run_cur = plsc.load_gather(run_v, [bi])
        lrank = run_cur + prefix
        plsc.store_scatter(run_v, [bi], run_cur + prefix + extra + 1,
                           mask=prefix == 0)
        eidx_v[pl.ds(tbase, L)] = bi
        lrank_v[pl.ds(tbase, L)] = lrank
        return 0

    lax.fori_loop(0, G, group_body, 0)

    pltpu.sync_copy(eidx_v, eidx_hbm.at[pl.ds(base, TPW)])
    pltpu.sync_copy(lrank_v, lrank_hbm.at[pl.ds(base, TPW)])
    pltpu.sync_copy(run_v, hist_hbm.at[wid])


@functools.partial(
    pl.kernel,
    mesh=_MESH,
    compiler_params=_CPARAMS,
    out_type=(
        jax.ShapeDtypeStruct((T, D), jnp.float32),
        jax.ShapeDtypeStruct((T,), jnp.int32),
        jax.ShapeDtypeStruct((E,), jnp.int32),
    ),
    scratch_types=[
        pltpu.VMEM((NW, E), jnp.int32),
        pltpu.VMEM((E,), jnp.int32),
        pltpu.VMEM((E,), jnp.int32),
        pltpu.VMEM((TPW,), jnp.int32),
        pltpu.VMEM((TPW,), jnp.int32),
        pltpu.VMEM((NCH, CH), jnp.int32),
        pltpu.VMEM((TPW,), jnp.int32),
        pltpu.VMEM((CH, D), jnp.float32),
        pltpu.VMEM((CH, D), jnp.float32),
        pltpu.SemaphoreType.DMA,
        pltpu.SemaphoreType.DMA,
    ],
)
def _dispatch(flow_hbm, eidx_hbm, lrank_hbm, hist_hbm,
              disp_hbm, order_hbm, counts_hbm,
              hist_v, gb_v, tot_v, eidx_v, lrank_v, rank_v, tokid_v,
              buf0, buf1, sem0, sem1):
    wid = _wid()
    base = wid * TPW
    pltpu.sync_copy(hist_hbm, hist_v)
    pltpu.sync_copy(eidx_hbm.at[pl.ds(base, TPW)], eidx_v)
    pltpu.sync_copy(lrank_hbm.at[pl.ds(base, TPW)], lrank_v)

    lane = lax.iota(jnp.int32, L)
    zeros = jnp.zeros((L,), jnp.int32)

    carry = jnp.int32(0)
    for q in range(E // L):
        tot = zeros
        tb = zeros
        for t in range(NW):
            h = hist_v[t, pl.ds(q * L, L)]
            tot = tot + h
            tb = tb + jnp.where(jnp.int32(t) < wid, h, zeros)
        inc = jnp.cumsum(tot)
        gb_v[pl.ds(q * L, L)] = (inc - tot + carry) + tb
        tot_v[pl.ds(q * L, L)] = tot
        carry = carry + jnp.sum(tot)

    @pl.when(wid == 0)
    def _():
        pltpu.sync_copy(tot_v, counts_hbm)

    for g in range(G):
        e = eidx_v[pl.ds(g * L, L)]
        rk = plsc.load_gather(gb_v, [e]) + lrank_v[pl.ds(g * L, L)]
        rank_v[g // (CH // L), pl.ds((g % (CH // L)) * L, L)] = rk
        tokid_v[pl.ds(g * L, L)] = base + g * L + lane

    pltpu.sync_copy(flow_hbm.at[pl.ds(base, CH), :], buf0)
    for c in range(NCH):
        buf, nbuf = (buf0, buf1) if c % 2 == 0 else (buf1, buf0)
        if c + 1 < NCH:
            ld = pltpu.async_copy(
                flow_hbm.at[pl.ds(base + (c + 1) * CH, CH), :], nbuf, sem1)
        st = pltpu.async_copy(buf, disp_hbm.at[rank_v.at[c]], sem0)
        pltpu.async_copy(tokid_v.at[pl.ds(c * CH, CH)],
                         order_hbm.at[rank_v.at[c]], sem0).wait()
        st.wait()
        if c + 1 < NCH:
            ld.wait()


def kernel(in_flow, score):
    eidx, lrank, hist = _routing(score.reshape(T * E))
    return _dispatch(in_flow, eidx, lrank, hist)
